# baseline (speedup 1.0000x reference)
import jax
import jax.numpy as jnp
from jax import lax
from jax.experimental import pallas as pl
from jax.experimental.pallas import tpu as pltpu
from jax.experimental.pallas import tpu_sc as plsc

B = 16384
C = 26
EMB = 256
L = 16
NC = 2
NS = 16
NW = NC * NS
S = 8192
ROWS = S // NW
HALF = 128
CH = 128
NCH = ROWS // CH
BR = 1024


def _sc_body(xt_hbm, out_hbm, vals8, buf0, buf1, sem0, sem1):
    wid = lax.axis_index("s") * NC + lax.axis_index("c")
    base = wid * ROWS
    pltpu.sync_copy(xt_hbm.at[pl.ds(0, 8), pl.ds(base, ROWS)], vals8)
    bufs = (buf0, buf1)
    sems = (sem0, sem1)
    pending = [(), ()]
    for c in range(NCH):
        buf = bufs[c % 2]
        for cp in pending[c % 2]:
            cp.wait()

        def group_body(g, c=c, buf=buf):
            v16 = vals8[0, pl.ds(c * CH + g * L, L)]
            for i in range(L):
                splat = jnp.full((L,), v16[i], jnp.int32)
                for j in range(HALF // L):
                    buf[g * L + i, pl.ds(j * L, L)] = splat

        plsc.parallel_loop(0, CH // L, unroll=2)(group_body)

        pending[c % 2] = tuple(
            pltpu.async_copy(
                buf,
                out_hbm.at[pl.ds(base + c * CH, CH), pl.ds(h * HALF, HALF)],
                sems[c % 2],
            )
            for h in range(EMB // HALF)
        )
    for grp in pending:
        for cp in grp:
            cp.wait()


def _tc_body(x_ref, part_ref, out_ref):
    del part_ref
    vals = x_ref[0:1, :]
    col = jnp.transpose(vals, (1, 0))
    out_ref[...] = jnp.broadcast_to(col, (BR, EMB))


def kernel(inputs):
    xt = inputs.T
    mesh = plsc.VectorSubcoreMesh(core_axis_name="c", subcore_axis_name="s")
    sc_fill = pl.kernel(
        _sc_body,
        out_type=jax.ShapeDtypeStruct((B, EMB), jnp.int32),
        mesh=mesh,
        scratch_types=[
            pltpu.VMEM((8, ROWS), jnp.int32),
            pltpu.VMEM((CH, HALF), jnp.int32),
            pltpu.VMEM((CH, HALF), jnp.int32),
            pltpu.SemaphoreType.DMA,
            pltpu.SemaphoreType.DMA,
        ],
    )
    part = sc_fill(xt)
    out = pl.pallas_call(
        _tc_body,
        grid=((B - S) // BR,),
        in_specs=[
            pl.BlockSpec((8, BR), lambda i: (0, (S // BR) + i)),
            pl.BlockSpec(memory_space=pl.ANY),
        ],
        out_specs=pl.BlockSpec((BR, EMB), lambda i: ((S // BR) + i, 0)),
        out_shape=jax.ShapeDtypeStruct((B, EMB), jnp.int32),
        input_output_aliases={1: 0},
    )(xt, part)
    return out


# speedup vs baseline: 1.1656x; 1.1656x over previous
import jax
import jax.numpy as jnp
from jax import lax
from jax.experimental import pallas as pl
from jax.experimental.pallas import tpu as pltpu
from jax.experimental.pallas import tpu_sc as plsc

B = 16384
C = 26
EMB = 256
L = 16
NC = 2
NS = 16
NW = NC * NS
ROWS = B // NW
HALF = 128
CH = 256
NCH = ROWS // CH


def _body(xt_hbm, out_hbm, vals8, buf0, buf1, sem0, sem1):
    wid = lax.axis_index("s") * NC + lax.axis_index("c")
    base = wid * ROWS
    pltpu.sync_copy(xt_hbm.at[pl.ds(0, 8), pl.ds(base, ROWS)], vals8)
    bufs = (buf0, buf1)
    sems = (sem0, sem1)
    pending = [(), ()]
    for c in range(NCH):
        buf = bufs[c % 2]
        for cp in pending[c % 2]:
            cp.wait()

        def group_body(g, c=c, buf=buf):
            v16 = vals8[0, pl.ds(c * CH + g * L, L)]
            for i in range(L):
                splat = jnp.full((L,), v16[i], jnp.int32)
                for j in range(HALF // L):
                    buf[g * L + i, pl.ds(j * L, L)] = splat

        plsc.parallel_loop(0, CH // L, unroll=2)(group_body)

        pending[c % 2] = tuple(
            pltpu.async_copy(
                buf,
                out_hbm.at[pl.ds(base + c * CH, CH), pl.ds(h * HALF, HALF)],
                sems[c % 2],
            )
            for h in range(EMB // HALF)
        )
    for grp in pending:
        for cp in grp:
            cp.wait()


def kernel(inputs):
    xt = inputs.T
    mesh = plsc.VectorSubcoreMesh(core_axis_name="c", subcore_axis_name="s")
    k = pl.kernel(
        _body,
        out_type=jax.ShapeDtypeStruct((B, EMB), jnp.int32),
        mesh=mesh,
        scratch_types=[
            pltpu.VMEM((8, ROWS), jnp.int32),
            pltpu.VMEM((CH, HALF), jnp.int32),
            pltpu.VMEM((CH, HALF), jnp.int32),
            pltpu.SemaphoreType.DMA,
            pltpu.SemaphoreType.DMA,
        ],
    )
    return k(xt)
